# baseline (speedup 1.0000x reference)
import functools

import jax
import jax.numpy as jnp
from jax import lax
from jax.experimental import pallas as pl
from jax.experimental.pallas import tpu as pltpu
from jax.experimental.pallas import tpu_sc as plsc

MAX_ROWS = 8192
EMB = 1024
NC = 2
NS = 16
NW = NC * NS
B_PER_W = MAX_ROWS // NW
CHUNK = 32
N_CHUNKS = B_PER_W // CHUNK
NBUF = 2


def _gather_body(table_hbm, idx_hbm, out_hbm, idx_v, buf0, buf1,
                 gsem0, gsem1, wsem0, wsem1):
    bufs = (buf0, buf1)
    gsems = (gsem0, gsem1)
    wsems = (wsem0, wsem1)
    wid = lax.axis_index("s") * NC + lax.axis_index("c")
    base = wid * B_PER_W
    pltpu.sync_copy(idx_hbm.at[pl.ds(base, B_PER_W)], idx_v)

    def gather(g):
        b = g % NBUF
        return pltpu.async_copy(
            table_hbm.at[idx_v.at[pl.ds(g * CHUNK, CHUNK)]], bufs[b], gsems[b]
        )

    def writeback(g):
        b = g % NBUF
        return pltpu.async_copy(
            bufs[b], out_hbm.at[pl.ds(base + g * CHUNK, CHUNK)], wsems[b]
        )

    gathers = [None] * N_CHUNKS
    writes = [None] * N_CHUNKS
    gathers[0] = gather(0)
    for g in range(N_CHUNKS):
        gathers[g].wait()
        if g + 1 < N_CHUNKS:
            if g - 1 >= 0:
                writes[g - 1].wait()
            gathers[g + 1] = gather(g + 1)
        writes[g] = writeback(g)
    writes[N_CHUNKS - 2].wait()
    writes[N_CHUNKS - 1].wait()


_sc_gather = functools.partial(
    pl.kernel,
    out_type=jax.ShapeDtypeStruct((MAX_ROWS, EMB), jnp.float32),
    mesh=plsc.VectorSubcoreMesh(core_axis_name="c", subcore_axis_name="s"),
    scratch_types=[
        pltpu.VMEM((B_PER_W,), jnp.int32),
        pltpu.VMEM((CHUNK, EMB), jnp.float32),
        pltpu.VMEM((CHUNK, EMB), jnp.float32),
        pltpu.SemaphoreType.DMA,
        pltpu.SemaphoreType.DMA,
        pltpu.SemaphoreType.DMA,
        pltpu.SemaphoreType.DMA,
    ],
)(_gather_body)


def kernel(seq_len, table):
    n = table.shape[0]
    offset = jnp.asarray(seq_len, dtype=jnp.int32) - jnp.int32(n)
    idx = jnp.clip(jnp.arange(n, dtype=jnp.int32) + offset, 0, n - 1)
    return _sc_gather(table, idx)
